# baseline (speedup 1.0000x reference)
import functools

import jax
import jax.numpy as jnp
from jax import lax
from jax.experimental import pallas as pl
from jax.experimental.pallas import tpu as pltpu
from jax.experimental.pallas import tpu_sc as plsc

_VOCAB = 1000
_NTOK = 16384
_CHUNK = 64


@functools.cache
def _build():
    info = plsc.get_sparse_core_info()
    nw = info.num_cores * info.num_subcores
    b_per_w = _NTOK // nw
    n_chunks = b_per_w // _CHUNK

    mesh = plsc.VectorSubcoreMesh(core_axis_name="c", subcore_axis_name="s")

    @functools.partial(
        pl.kernel,
        out_type=jax.ShapeDtypeStruct((_NTOK, _VOCAB), jnp.float32),
        mesh=mesh,
        compiler_params=pltpu.CompilerParams(use_tc_tiling_on_sc=False),
        scratch_types=[
            pltpu.VMEM((b_per_w,), jnp.int32),
            pltpu.VMEM((_CHUNK, _VOCAB), jnp.float32),
            pltpu.VMEM((_CHUNK, _VOCAB), jnp.float32),
            pltpu.SemaphoreType.DMA,
            pltpu.SemaphoreType.DMA,
            pltpu.SemaphoreType.DMA,
            pltpu.SemaphoreType.DMA,
        ],
    )
    def emb_kernel(x_hbm, table_hbm, out_hbm, idx_v, rows0, rows1,
                   gsem0, gsem1, ssem0, ssem1):
        wid = lax.axis_index("s") * info.num_cores + lax.axis_index("c")
        base = wid * b_per_w
        pltpu.sync_copy(x_hbm.at[pl.ds(base, b_per_w)], idx_v)

        bufs = (rows0, rows1)
        gsems = (gsem0, gsem1)
        ssems = (ssem0, ssem1)
        gather = [None, None]
        store = [None, None]

        gather[0] = pltpu.async_copy(
            table_hbm.at[idx_v.at[pl.ds(0, _CHUNK)]], bufs[0], gsems[0])
        for g in range(n_chunks):
            cur = g & 1
            nxt = 1 - cur
            if g + 1 < n_chunks:
                if store[nxt] is not None:
                    store[nxt].wait()
                gather[nxt] = pltpu.async_copy(
                    table_hbm.at[idx_v.at[pl.ds((g + 1) * _CHUNK, _CHUNK)]],
                    bufs[nxt], gsems[nxt])
            gather[cur].wait()
            store[cur] = pltpu.async_copy(
                bufs[cur], out_hbm.at[pl.ds(base + g * _CHUNK, _CHUNK)],
                ssems[cur])
        store[0].wait()
        store[1].wait()

    return emb_kernel


def kernel(x, table):
    return _build()(x, table)
